# baseline (speedup 1.0000x reference)
import functools

import jax
import jax.numpy as jnp
from jax.experimental import pallas as pl
from jax.experimental.pallas import tpu as pltpu

BN = 512


def _stats(sx, sxx, n):
    gm = sx * (1.0 / n)
    gv = (sxx - sx * gm) * (1.0 / (n - 1))
    return gv * 0.05 - gm * 0.1


def _main_kernel(n, x_ref, rmt_ref, bias_ref, proc_ref, d0_ref, d1_ref,
                 sx1, sxx1, sx2, sxx2):
    ph = pl.program_id(1)
    nb = pl.program_id(2)
    d = x_ref.shape[-1]
    p = bias_ref.shape[-1]

    @pl.when(ph == 0)
    def _phase0():
        x = x_ref[0]
        ps = jnp.sum(x, axis=0, keepdims=True)
        pss = jnp.sum(x * x, axis=0, keepdims=True)

        @pl.when(nb == 0)
        def _():
            sx1[...] = ps
            sxx1[...] = pss

        @pl.when(nb != 0)
        def _():
            sx1[...] += ps
            sxx1[...] += pss

    @pl.when(ph == 1)
    def _phase1():
        w1 = _stats(sx1[...], sxx1[...], n)
        y = x_ref[0] * 1.1 + w1
        nrm2 = jnp.mean(y * y, axis=1, keepdims=True) + 1e-6
        x1 = y * (1.0 / jnp.maximum(jnp.sqrt(nrm2), 1.0))
        ps = jnp.sum(x1, axis=0, keepdims=True)
        pss = jnp.sum(x1 * x1, axis=0, keepdims=True)

        @pl.when(nb == 0)
        def _():
            sx2[...] = ps
            sxx2[...] = pss

        @pl.when(nb != 0)
        def _():
            sx2[...] += ps
            sxx2[...] += pss

    @pl.when(ph == 2)
    def _phase2():
        w1 = _stats(sx1[...], sxx1[...], n)
        w2 = _stats(sx2[...], sxx2[...], n)
        w2_f = jnp.mean(w2[:, : d // 2], axis=1, keepdims=True)
        w2_s = jnp.mean(w2[:, d // 2:], axis=1, keepdims=True)
        w2sq = jnp.mean(w2 * w2, axis=1, keepdims=True)

        y = x_ref[0] * 1.1 + w1
        m_yy = jnp.mean(y * y, axis=1, keepdims=True)
        m_yw = jnp.mean(y * w2, axis=1, keepdims=True)
        my_f = jnp.mean(y[:, : d // 2], axis=1, keepdims=True)
        my_s = jnp.mean(y[:, d // 2:], axis=1, keepdims=True)

        inv1 = 1.0 / jnp.maximum(jnp.sqrt(m_yy + 1e-6), 1.0)
        m_y2y2 = (1.21 * (inv1 * inv1)) * m_yy + (2.2 * inv1) * m_yw + w2sq
        inv2 = 1.0 / jnp.maximum(jnp.sqrt(m_y2y2 + 1e-6), 1.0)
        a = (1.1 * inv1) * inv2

        mu_f0 = a * my_f + inv2 * w2_f
        mu_s0 = a * my_s + inv2 * w2_s
        mu = (mu_f0 + mu_s0) * 0.5
        q = (a * a) * m_yy + (2.0 * a * inv2) * m_yw + (inv2 * inv2) * w2sq
        var = q - mu * mu

        Bc = jnp.zeros_like(mu)
        S = jnp.ones_like(mu)
        for i in range(3):
            c = (mu * 0.15 - var * 0.05) * (0.3 + 0.1 * i)
            q = q + (2.0 * c) * mu + c * c
            mu = mu + c
            Bc = Bc + c * S
            s = jnp.maximum(jnp.sqrt(q + 1e-6), 1.0)
            inv = 1.0 / s
            mu = mu * inv
            q = q * (inv * inv)
            var = var * (inv * inv)
            S = S * s

        invS = 1.0 / S
        mean_all = mu
        var_all = var
        mean_first = (mu_f0 + Bc) * invS
        mean_second = (mu_s0 + Bc) * invS

        x5 = y * (a * invS) + (w2 * (inv2 * invS) + Bc)
        absmean = jnp.mean(jnp.abs(x5), axis=1, keepdims=True)

        dm = jax.lax.dot_general(x5.astype(jnp.bfloat16),
                                 rmt_ref[...].astype(jnp.bfloat16),
                                 (((1,), (0,)), ((), ())),
                                 preferred_element_type=jnp.float32)
        logits = dm + bias_ref[...]

        best = logits[:, 0:1]
        sel = jnp.zeros_like(best, dtype=jnp.int32)
        for j in range(1, p):
            lj = logits[:, j:j + 1]
            upd = lj > best
            best = jnp.where(upd, lj, best)
            sel = jnp.where(upd, j, sel)

        scores = [mean_all, mean_first, mean_second, var_all, -absmean]
        for j in range(5, p):
            scores.append(mean_all * (1.0 + 0.05 * j) - 0.1 * var_all)
        proc = scores[0]
        for j in range(1, p):
            proc = jnp.where(sel == j, scores[j], proc)

        proc_ref[0] = proc
        d0_ref[0] = x5[:, 0:1]
        d1_ref[0] = x5[:, 1:2]


def _fin_kernel(k, proc_ref, d0_ref, d1_ref, cs_ref, sl_ref, idx_ref, sc_ref):
    b, n = proc_ref.shape
    nc = cs_ref.shape[1]
    cl = jnp.mean(cs_ref[...], axis=2)
    cm = jnp.mean(cl, axis=1, keepdims=True)
    cv = jnp.sum((cl - cm) ** 2, axis=1, keepdims=True) * (1.0 / (nc - 1))
    out = proc_ref[...] + (cm * -0.05 - cv * 0.02)
    ma = jnp.max(jnp.abs(out), axis=1, keepdims=True) + 1e-6
    out = out / jnp.maximum(ma, 1.0)
    delta = d0_ref[...] * 0.05 + d1_ref[...] * 0.03 + sl_ref[:, 0:1] * 0.01
    state = out
    for _ in range(8):
        state = state + delta
    iota = jax.lax.broadcasted_iota(jnp.int32, (b, n), 1)
    for j in range(k):
        m = jnp.max(state, axis=1, keepdims=True)
        idx = jnp.min(jnp.where(state == m, iota, n), axis=1, keepdims=True)
        sc_ref[:, j:j + 1] = m
        idx_ref[:, j:j + 1] = idx
        state = jnp.where(iota == idx, -jnp.inf, state)


def kernel(process_feats, core_states, sys_load, routing_matrix, bias):
    x = process_feats
    b, n, d = x.shape
    p = routing_matrix.shape[0]
    nb = n // BN
    f32 = jnp.float32

    x_spec = pl.BlockSpec((1, BN, d), lambda i, ph, j: (i, j, 0))
    col = jax.ShapeDtypeStruct((b, n, 1), f32)
    col_spec = pl.BlockSpec((1, BN, 1), lambda i, ph, j: (i, j, 0))

    proc, d0, d1 = pl.pallas_call(
        functools.partial(_main_kernel, n),
        grid=(b, 3, nb),
        in_specs=[x_spec,
                  pl.BlockSpec((d, p), lambda i, ph, j: (0, 0)),
                  pl.BlockSpec((1, p), lambda i, ph, j: (0, 0))],
        out_specs=[col_spec, col_spec, col_spec],
        out_shape=[col, col, col],
        scratch_shapes=[pltpu.VMEM((1, d), f32)] * 4,
        compiler_params=pltpu.CompilerParams(
            dimension_semantics=("arbitrary", "arbitrary", "arbitrary")),
    )(x, routing_matrix.T, bias.reshape(1, p))

    k = min(core_states.shape[1], n)
    idx, sc = pl.pallas_call(
        functools.partial(_fin_kernel, k),
        in_specs=[
            pl.BlockSpec((b, n), lambda: (0, 0)),
            pl.BlockSpec((b, n), lambda: (0, 0)),
            pl.BlockSpec((b, n), lambda: (0, 0)),
            pl.BlockSpec(core_states.shape, lambda: (0, 0, 0)),
            pl.BlockSpec(sys_load.shape, lambda: (0, 0)),
        ],
        out_specs=[pl.BlockSpec((b, k), lambda: (0, 0)),
                   pl.BlockSpec((b, k), lambda: (0, 0))],
        out_shape=[jax.ShapeDtypeStruct((b, k), jnp.int32),
                   jax.ShapeDtypeStruct((b, k), f32)],
    )(proc[..., 0], d0[..., 0], d1[..., 0], core_states, sys_load)
    return idx, sc


# speedup vs baseline: 1.0703x; 1.0703x over previous
import functools

import jax
import jax.numpy as jnp
from jax.experimental import pallas as pl
from jax.experimental.pallas import tpu as pltpu

BN = 512


def _stats(sx, sxx, n):
    gm = sx * (1.0 / n)
    gv = (sxx - sx * gm) * (1.0 / (n - 1))
    return gv * 0.05 - gm * 0.1


def _main_kernel(n, x_ref, rmt_ref, bias_ref, proc_ref, d0_ref, d1_ref,
                 x_scr, sx1, sxx1, sx2, sxx2):
    ph = pl.program_id(1)
    nb = pl.program_id(2)
    d = x_ref.shape[-1]
    p = bias_ref.shape[-1]

    @pl.when(ph == 0)
    def _phase0():
        x = x_ref[0]
        x_scr[pl.ds(nb * BN, BN), :] = x
        ps = jnp.sum(x, axis=0, keepdims=True)
        pss = jnp.sum(x * x, axis=0, keepdims=True)

        @pl.when(nb == 0)
        def _():
            sx1[...] = ps
            sxx1[...] = pss

        @pl.when(nb != 0)
        def _():
            sx1[...] += ps
            sxx1[...] += pss

    @pl.when(ph == 1)
    def _phase1():
        w1 = _stats(sx1[...], sxx1[...], n)
        y = x_scr[pl.ds(nb * BN, BN), :] * 1.1 + w1
        nrm2 = jnp.mean(y * y, axis=1, keepdims=True) + 1e-6
        x1 = y * (1.0 / jnp.maximum(jnp.sqrt(nrm2), 1.0))
        ps = jnp.sum(x1, axis=0, keepdims=True)
        pss = jnp.sum(x1 * x1, axis=0, keepdims=True)

        @pl.when(nb == 0)
        def _():
            sx2[...] = ps
            sxx2[...] = pss

        @pl.when(nb != 0)
        def _():
            sx2[...] += ps
            sxx2[...] += pss

    @pl.when(ph == 2)
    def _phase2():
        w1 = _stats(sx1[...], sxx1[...], n)
        w2 = _stats(sx2[...], sxx2[...], n)
        w2_f = jnp.mean(w2[:, : d // 2], axis=1, keepdims=True)
        w2_s = jnp.mean(w2[:, d // 2:], axis=1, keepdims=True)
        w2sq = jnp.mean(w2 * w2, axis=1, keepdims=True)

        y = x_scr[pl.ds(nb * BN, BN), :] * 1.1 + w1
        m_yy = jnp.mean(y * y, axis=1, keepdims=True)
        m_yw = jnp.mean(y * w2, axis=1, keepdims=True)
        my_f = jnp.mean(y[:, : d // 2], axis=1, keepdims=True)
        my_s = jnp.mean(y[:, d // 2:], axis=1, keepdims=True)

        inv1 = 1.0 / jnp.maximum(jnp.sqrt(m_yy + 1e-6), 1.0)
        m_y2y2 = (1.21 * (inv1 * inv1)) * m_yy + (2.2 * inv1) * m_yw + w2sq
        inv2 = 1.0 / jnp.maximum(jnp.sqrt(m_y2y2 + 1e-6), 1.0)
        a = (1.1 * inv1) * inv2

        mu_f0 = a * my_f + inv2 * w2_f
        mu_s0 = a * my_s + inv2 * w2_s
        mu = (mu_f0 + mu_s0) * 0.5
        q = (a * a) * m_yy + (2.0 * a * inv2) * m_yw + (inv2 * inv2) * w2sq
        var = q - mu * mu

        Bc = jnp.zeros_like(mu)
        S = jnp.ones_like(mu)
        for i in range(3):
            c = (mu * 0.15 - var * 0.05) * (0.3 + 0.1 * i)
            q = q + (2.0 * c) * mu + c * c
            mu = mu + c
            Bc = Bc + c * S
            s = jnp.maximum(jnp.sqrt(q + 1e-6), 1.0)
            inv = 1.0 / s
            mu = mu * inv
            q = q * (inv * inv)
            var = var * (inv * inv)
            S = S * s

        invS = 1.0 / S
        mean_all = mu
        var_all = var
        mean_first = (mu_f0 + Bc) * invS
        mean_second = (mu_s0 + Bc) * invS

        x5 = y * (a * invS) + (w2 * (inv2 * invS) + Bc)
        absmean = jnp.mean(jnp.abs(x5), axis=1, keepdims=True)

        dm = jax.lax.dot_general(x5.astype(jnp.bfloat16),
                                 rmt_ref[...].astype(jnp.bfloat16),
                                 (((1,), (0,)), ((), ())),
                                 preferred_element_type=jnp.float32)
        logits = dm + bias_ref[...]

        best = logits[:, 0:1]
        sel = jnp.zeros_like(best, dtype=jnp.int32)
        for j in range(1, p):
            lj = logits[:, j:j + 1]
            upd = lj > best
            best = jnp.where(upd, lj, best)
            sel = jnp.where(upd, j, sel)

        scores = [mean_all, mean_first, mean_second, var_all, -absmean]
        for j in range(5, p):
            scores.append(mean_all * (1.0 + 0.05 * j) - 0.1 * var_all)
        proc = scores[0]
        for j in range(1, p):
            proc = jnp.where(sel == j, scores[j], proc)

        proc_ref[0] = proc
        d0_ref[0] = x5[:, 0:1]
        d1_ref[0] = x5[:, 1:2]


def _fin_kernel(k, proc_ref, d0_ref, d1_ref, cs_ref, sl_ref, idx_ref, sc_ref):
    b, n = proc_ref.shape
    nc = cs_ref.shape[1]
    cl = jnp.mean(cs_ref[...], axis=2)
    cm = jnp.mean(cl, axis=1, keepdims=True)
    cv = jnp.sum((cl - cm) ** 2, axis=1, keepdims=True) * (1.0 / (nc - 1))
    out = proc_ref[...] + (cm * -0.05 - cv * 0.02)
    ma = jnp.max(jnp.abs(out), axis=1, keepdims=True) + 1e-6
    out = out / jnp.maximum(ma, 1.0)
    delta = d0_ref[...] * 0.05 + d1_ref[...] * 0.03 + sl_ref[:, 0:1] * 0.01
    state = out
    for _ in range(8):
        state = state + delta
    iota = jax.lax.broadcasted_iota(jnp.int32, (b, n), 1)
    for j in range(k):
        m = jnp.max(state, axis=1, keepdims=True)
        idx = jnp.min(jnp.where(state == m, iota, n), axis=1, keepdims=True)
        sc_ref[:, j:j + 1] = m
        idx_ref[:, j:j + 1] = idx
        state = jnp.where(iota == idx, -jnp.inf, state)


def kernel(process_feats, core_states, sys_load, routing_matrix, bias):
    x = process_feats
    b, n, d = x.shape
    p = routing_matrix.shape[0]
    nb = n // BN
    f32 = jnp.float32

    x_spec = pl.BlockSpec(
        (1, BN, d),
        lambda i, ph, j: (i, jax.lax.select(ph == 0, j, nb - 1), 0))
    col = jax.ShapeDtypeStruct((b, n, 1), f32)
    col_spec = pl.BlockSpec((1, BN, 1), lambda i, ph, j: (i, j, 0))

    proc, d0, d1 = pl.pallas_call(
        functools.partial(_main_kernel, n),
        grid=(b, 3, nb),
        in_specs=[x_spec,
                  pl.BlockSpec((d, p), lambda i, ph, j: (0, 0)),
                  pl.BlockSpec((1, p), lambda i, ph, j: (0, 0))],
        out_specs=[col_spec, col_spec, col_spec],
        out_shape=[col, col, col],
        scratch_shapes=[pltpu.VMEM((n, d), f32)] + [pltpu.VMEM((1, d), f32)] * 4,
        compiler_params=pltpu.CompilerParams(
            dimension_semantics=("arbitrary", "arbitrary", "arbitrary")),
    )(x, routing_matrix.T, bias.reshape(1, p))

    k = min(core_states.shape[1], n)
    idx, sc = pl.pallas_call(
        functools.partial(_fin_kernel, k),
        in_specs=[
            pl.BlockSpec((b, n), lambda: (0, 0)),
            pl.BlockSpec((b, n), lambda: (0, 0)),
            pl.BlockSpec((b, n), lambda: (0, 0)),
            pl.BlockSpec(core_states.shape, lambda: (0, 0, 0)),
            pl.BlockSpec(sys_load.shape, lambda: (0, 0)),
        ],
        out_specs=[pl.BlockSpec((b, k), lambda: (0, 0)),
                   pl.BlockSpec((b, k), lambda: (0, 0))],
        out_shape=[jax.ShapeDtypeStruct((b, k), jnp.int32),
                   jax.ShapeDtypeStruct((b, k), f32)],
    )(proc[..., 0], d0[..., 0], d1[..., 0], core_states, sys_load)
    return idx, sc
